# baseline (speedup 1.0000x reference)
import functools

import jax
import jax.numpy as jnp
from jax import lax
from jax.experimental import pallas as pl
from jax.experimental.pallas import tpu as pltpu
from jax.experimental.pallas import tpu_sc as plsc

_NC, _NS = 2, 16
_NW = _NC * _NS
_L16 = 16
_CHUNK = 800
_PK = 320


@functools.cache
def _build_pack(v, d):
    n_blocks = v // _PK
    mesh = plsc.VectorSubcoreMesh(core_axis_name="c", subcore_axis_name="s")
    blocks_per_w = (n_blocks + _NW - 1) // _NW

    @functools.partial(
        pl.kernel,
        out_type=jax.ShapeDtypeStruct((v // 2, 2 * d), jnp.float32),
        mesh=mesh,
        scratch_types=[
            pltpu.VMEM((_PK, d), jnp.float32),
            pltpu.VMEM((_PK, d), jnp.float32),
            pltpu.VMEM((_PK // 2, 2 * d), jnp.float32),
            pltpu.VMEM((_PK // 2, 2 * d), jnp.float32),
            pltpu.SemaphoreType.DMA,
            pltpu.SemaphoreType.DMA,
            pltpu.SemaphoreType.DMA,
            pltpu.SemaphoreType.DMA,
        ],
    )
    def pack(table_hbm, t2_hbm, in0, in1, out0, out1,
             rsem0, rsem1, wsem0, wsem1):
        wid = lax.axis_index("s") * _NC + lax.axis_index("c")
        in_v = (in0, in1)
        out_v = (out0, out1)
        rsem = (rsem0, rsem1)
        wsem = (wsem0, wsem1)

        def blk_id(i):
            return wid + i * _NW

        def read_start(i, u):
            pltpu.async_copy(
                table_hbm.at[pl.ds(blk_id(i) * _PK, _PK)], in_v[u], rsem[u])

        def read_wait(u):
            pltpu.make_async_copy(
                table_hbm.at[pl.ds(0, _PK)], in_v[u], rsem[u]).wait()

        def repack(u):
            @pl.loop(0, _PK // 2)
            def _(k):
                for g in range(d // _L16):
                    out_v[u][k, pl.ds(g * _L16, _L16)] = (
                        in_v[u][2 * k, pl.ds(g * _L16, _L16)])
                    out_v[u][k, pl.ds(d + g * _L16, _L16)] = (
                        in_v[u][2 * k + 1, pl.ds(g * _L16, _L16)])

        def write_start(i, u):
            pltpu.async_copy(
                out_v[u], t2_hbm.at[pl.ds(blk_id(i) * (_PK // 2), _PK // 2)],
                wsem[u])

        def write_wait(u):
            pltpu.make_async_copy(
                out_v[u], t2_hbm.at[pl.ds(0, _PK // 2)], wsem[u]).wait()

        def valid(i):
            return blk_id(i) < n_blocks

        @pl.when(valid(0))
        def _():
            read_start(0, 0)

        @pl.when(valid(1))
        def _():
            read_start(1, 1)

        @pl.loop(0, (blocks_per_w + 1) // 2)
        def _outer(j):
            for u in (0, 1):
                i = j * 2 + u

                @pl.when(valid(i))
                def _():
                    read_wait(u)

                    @pl.when(i >= 2)
                    def _():
                        write_wait(u)
                    repack(u)
                    write_start(i, u)

                    @pl.when(valid(i + 2))
                    def _():
                        read_start(i + 2, u)

        @pl.when(valid(0))
        def _():
            write_wait(0)

        @pl.when(valid(1))
        def _():
            write_wait(1)

    return pack


@functools.cache
def _build_gather(n, d):
    n_per_w = n // _NW
    n_chunks = n_per_w // _CHUNK
    assert n_chunks % 2 == 0 and n_chunks >= 4
    mesh = plsc.VectorSubcoreMesh(core_axis_name="c", subcore_axis_name="s")

    @functools.partial(
        pl.kernel,
        out_type=jax.ShapeDtypeStruct((n, d), jnp.float32),
        mesh=mesh,
        scratch_types=[
            pltpu.VMEM((_CHUNK,), jnp.int32),
            pltpu.VMEM((_CHUNK,), jnp.int32),
            pltpu.VMEM((_CHUNK, d), jnp.float32),
            pltpu.VMEM((_CHUNK, d), jnp.float32),
            pltpu.SemaphoreType.DMA,
            pltpu.SemaphoreType.DMA,
            pltpu.SemaphoreType.DMA,
            pltpu.SemaphoreType.DMA,
            pltpu.SemaphoreType.DMA,
            pltpu.SemaphoreType.DMA,
        ],
        compiler_params=pltpu.CompilerParams(use_tc_tiling_on_sc=False),
    )
    def gather(idx_hbm, table_hbm, out_hbm,
               idx0, idx1, rows0, rows1,
               isem0, isem1, gsem0, gsem1, wsem0, wsem1):
        wid = lax.axis_index("s") * _NC + lax.axis_index("c")
        base = wid * n_per_w
        idx_v = (idx0, idx1)
        rows_v = (rows0, rows1)
        isem = (isem0, isem1)
        gsem = (gsem0, gsem1)
        wsem = (wsem0, wsem1)

        def idx_start(i, u):
            pltpu.async_copy(
                idx_hbm.at[pl.ds(base + i * _CHUNK, _CHUNK)], idx_v[u],
                isem[u])

        def idx_wait(u):
            pltpu.make_async_copy(
                idx_hbm.at[pl.ds(0, _CHUNK)], idx_v[u], isem[u]).wait()

        def gather_start(u):
            pltpu.async_copy(table_hbm.at[idx_v[u]], rows_v[u], gsem[u])

        def gather_wait(u):
            pltpu.make_async_copy(
                table_hbm.at[idx_v[u]], rows_v[u], gsem[u]).wait()

        def write_start(i, u):
            pltpu.async_copy(
                rows_v[u], out_hbm.at[pl.ds(base + i * _CHUNK, _CHUNK)],
                wsem[u])

        def write_wait(u):
            pltpu.make_async_copy(
                rows_v[u], out_hbm.at[pl.ds(0, _CHUNK)], wsem[u]).wait()

        idx_start(0, 0)
        idx_start(1, 1)
        idx_wait(0)
        gather_start(0)

        @pl.loop(0, n_chunks // 2)
        def _outer(j):
            for u in (0, 1):
                i = j * 2 + u
                nu = 1 - u
                gather_wait(u)

                @pl.when(i + 2 < n_chunks)
                def _():
                    idx_start(i + 2, u)

                @pl.when(i + 1 < n_chunks)
                def _():
                    idx_wait(nu)

                    @pl.when(i >= 1)
                    def _():
                        write_wait(nu)
                    gather_start(nu)

                write_start(i, u)

        write_wait(0)
        write_wait(1)

    return gather


@functools.cache
def _build_writer(b, l, d):
    grp_per_w = b // 2 // _NW
    hl = l // 2
    mesh = plsc.VectorSubcoreMesh(core_axis_name="c", subcore_axis_name="s")

    @functools.partial(
        pl.kernel,
        out_type=jax.ShapeDtypeStruct((b, l, d), jnp.float32),
        mesh=mesh,
        scratch_types=[
            pltpu.VMEM((l, 2 * d), jnp.float32),
            pltpu.VMEM((l, 2 * d), jnp.float32),
            pltpu.VMEM((l, d), jnp.float32),
            pltpu.VMEM((l, d), jnp.float32),
            pltpu.SemaphoreType.DMA,
            pltpu.SemaphoreType.DMA,
            pltpu.SemaphoreType.DMA,
            pltpu.SemaphoreType.DMA,
        ],
    )
    def writer(packed_hbm, out_hbm, in0, in1, sela, selb,
               rsem0, rsem1, wsema, wsemb):
        wid = lax.axis_index("s") * _NC + lax.axis_index("c")
        grp0 = wid * grp_per_w
        in_v = (in0, in1)
        sel_v = (sela, selb)
        rsem = (rsem0, rsem1)
        wsem = (wsema, wsemb)

        def read_start(i, u):
            pltpu.async_copy(packed_hbm.at[grp0 + i], in_v[u], rsem[u])

        def read_wait(u):
            pltpu.make_async_copy(
                packed_hbm.at[0], in_v[u], rsem[u]).wait()

        def unpack(u, s):
            @pl.loop(0, hl)
            def _(k):
                for g in range(d // _L16):
                    sel_v[s][2 * k, pl.ds(g * _L16, _L16)] = (
                        in_v[u][s * hl + k, pl.ds(g * _L16, _L16)])
                    sel_v[s][2 * k + 1, pl.ds(g * _L16, _L16)] = (
                        in_v[u][s * hl + k, pl.ds(d + g * _L16, _L16)])

        def write_start(i, s):
            pltpu.async_copy(
                sel_v[s], out_hbm.at[2 * (grp0 + i) + s], wsem[s])

        def write_wait(s):
            pltpu.make_async_copy(sel_v[s], out_hbm.at[0], wsem[s]).wait()

        read_start(0, 0)
        read_start(1, 1)

        @pl.loop(0, grp_per_w // 2)
        def _outer(j):
            for u in (0, 1):
                i = j * 2 + u
                read_wait(u)
                for s in (0, 1):
                    @pl.when(i >= 1)
                    def _():
                        write_wait(s)
                    unpack(u, s)
                    write_start(i, s)

                @pl.when(i + 2 < grp_per_w)
                def _():
                    read_start(i + 2, u)

        write_wait(0)
        write_wait(1)

    return writer


def kernel(sequences_tokens, embedding_table):
    b, l = sequences_tokens.shape
    v, d = embedding_table.shape
    idx = sequences_tokens.reshape(b * l)
    t2 = _build_pack(v, d)(embedding_table)
    t_view = t2.reshape(v, d)
    packed = _build_gather(b * l, d)(idx, t_view)
    packed3 = packed.reshape(b // 2, l, 2 * d)
    return _build_writer(b, l, d)(packed3)


# speedup vs baseline: 1.2832x; 1.2832x over previous
import functools

import jax
import jax.numpy as jnp
from jax import lax
from jax.experimental import pallas as pl
from jax.experimental.pallas import tpu as pltpu
from jax.experimental.pallas import tpu_sc as plsc

_NC, _NS = 2, 16
_NW = _NC * _NS
_CHUNK = 800


@functools.cache
def _build_gather(n, d):
    n_per_w = n // _NW
    n_chunks = n_per_w // _CHUNK
    assert n_chunks % 2 == 0 and n_chunks >= 4
    mesh = plsc.VectorSubcoreMesh(core_axis_name="c", subcore_axis_name="s")

    @functools.partial(
        pl.kernel,
        out_type=jax.ShapeDtypeStruct((n, d), jnp.float32),
        mesh=mesh,
        scratch_types=[
            pltpu.VMEM((_CHUNK,), jnp.int32),
            pltpu.VMEM((_CHUNK,), jnp.int32),
            pltpu.VMEM((_CHUNK, d), jnp.float32),
            pltpu.VMEM((_CHUNK, d), jnp.float32),
            pltpu.SemaphoreType.DMA,
            pltpu.SemaphoreType.DMA,
            pltpu.SemaphoreType.DMA,
            pltpu.SemaphoreType.DMA,
            pltpu.SemaphoreType.DMA,
            pltpu.SemaphoreType.DMA,
        ],
        compiler_params=pltpu.CompilerParams(use_tc_tiling_on_sc=False),
    )
    def gather(idx_hbm, table_hbm, out_hbm,
               idx0, idx1, rows0, rows1,
               isem0, isem1, gsem0, gsem1, wsem0, wsem1):
        wid = lax.axis_index("s") * _NC + lax.axis_index("c")
        base = wid * n_per_w
        idx_v = (idx0, idx1)
        rows_v = (rows0, rows1)
        isem = (isem0, isem1)
        gsem = (gsem0, gsem1)
        wsem = (wsem0, wsem1)

        def idx_start(i, u):
            pltpu.async_copy(
                idx_hbm.at[pl.ds(base + i * _CHUNK, _CHUNK)], idx_v[u],
                isem[u])

        def idx_wait(u):
            pltpu.make_async_copy(
                idx_hbm.at[pl.ds(0, _CHUNK)], idx_v[u], isem[u]).wait()

        def gather_start(u):
            pltpu.async_copy(table_hbm.at[idx_v[u]], rows_v[u], gsem[u])

        def gather_wait(u):
            pltpu.make_async_copy(
                table_hbm.at[idx_v[u]], rows_v[u], gsem[u]).wait()

        def write_start(i, u):
            pltpu.async_copy(
                rows_v[u], out_hbm.at[pl.ds(base + i * _CHUNK, _CHUNK)],
                wsem[u])

        def write_wait(u):
            pltpu.make_async_copy(
                rows_v[u], out_hbm.at[pl.ds(0, _CHUNK)], wsem[u]).wait()

        idx_start(0, 0)
        idx_start(1, 1)
        idx_wait(0)
        gather_start(0)

        @pl.loop(0, n_chunks // 2)
        def _outer(j):
            for u in (0, 1):
                i = j * 2 + u
                nu = 1 - u
                gather_wait(u)

                @pl.when(i + 2 < n_chunks)
                def _():
                    idx_start(i + 2, u)

                @pl.when(i + 1 < n_chunks)
                def _():
                    idx_wait(nu)

                    @pl.when(i >= 1)
                    def _():
                        write_wait(nu)
                    gather_start(nu)

                write_start(i, u)

        write_wait(0)
        write_wait(1)

    return gather


def kernel(sequences_tokens, embedding_table):
    b, l = sequences_tokens.shape
    _, d = embedding_table.shape
    idx = sequences_tokens.reshape(b * l)
    out = _build_gather(b * l, d)(idx, embedding_table)
    return out.reshape(b, l, d)


# speedup vs baseline: 1.5472x; 1.2057x over previous
import functools

import jax
import jax.numpy as jnp
from jax import lax
from jax.experimental import pallas as pl
from jax.experimental.pallas import tpu as pltpu
from jax.experimental.pallas import tpu_sc as plsc

_NC, _NS = 2, 16
_NW = _NC * _NS
_CHUNK = 800


@functools.cache
def _build_gather(n, d):
    n_per_w = n // _NW
    n_chunks = n_per_w // _CHUNK
    assert n_chunks % 2 == 0 and n_chunks >= 4
    mesh = plsc.VectorSubcoreMesh(core_axis_name="c", subcore_axis_name="s")

    @functools.partial(
        pl.kernel,
        out_type=jax.ShapeDtypeStruct((n, d), jnp.float32),
        mesh=mesh,
        scratch_types=[
            pltpu.VMEM((_CHUNK,), jnp.int32),
            pltpu.VMEM((_CHUNK,), jnp.int32),
            pltpu.VMEM((_CHUNK, d), jnp.float32),
            pltpu.VMEM((_CHUNK, d), jnp.float32),
            pltpu.SemaphoreType.DMA,
            pltpu.SemaphoreType.DMA,
            pltpu.SemaphoreType.DMA,
            pltpu.SemaphoreType.DMA,
            pltpu.SemaphoreType.DMA,
            pltpu.SemaphoreType.DMA,
        ],
        compiler_params=pltpu.CompilerParams(use_tc_tiling_on_sc=False),
    )
    def gather(idx_hbm, table_hbm, out_hbm,
               idx0, idx1, rows0, rows1,
               isem0, isem1, gsem0, gsem1, wsem0, wsem1):
        wid = lax.axis_index("s") * _NC + lax.axis_index("c")
        base = wid * n_per_w
        idx_v = (idx0, idx1)
        rows_v = (rows0, rows1)
        isem = (isem0, isem1)
        gsem = (gsem0, gsem1)
        wsem = (wsem0, wsem1)

        def idx_start(i, u):
            pltpu.async_copy(
                idx_hbm.at[pl.ds(base + i * _CHUNK, _CHUNK)], idx_v[u],
                isem[u])

        def idx_wait(u):
            pltpu.make_async_copy(
                idx_hbm.at[pl.ds(0, _CHUNK)], idx_v[u], isem[u]).wait()

        def gather_start(u):
            pltpu.async_copy(table_hbm.at[idx_v[u]], rows_v[u], gsem[u])

        def gather_wait(u):
            pltpu.make_async_copy(
                table_hbm.at[idx_v[u]], rows_v[u], gsem[u]).wait()

        def write_start(i, u):
            pltpu.async_copy(
                rows_v[u], out_hbm.at[pl.ds(base + i * _CHUNK, _CHUNK)],
                wsem[u])

        def write_wait(u):
            pltpu.make_async_copy(
                rows_v[u], out_hbm.at[pl.ds(0, _CHUNK)], wsem[u]).wait()

        idx_start(0, 0)
        idx_start(1, 1)
        idx_wait(0)
        gather_start(0)

        @pl.loop(0, n_chunks // 2)
        def _outer(j):
            for u in (0, 1):
                i = j * 2 + u
                nu = 1 - u
                gather_wait(u)

                @pl.when(i + 2 < n_chunks)
                def _():
                    idx_start(i + 2, u)

                @pl.when(i + 1 < n_chunks)
                def _():
                    idx_wait(nu)

                    @pl.when(i >= 1)
                    def _():
                        write_wait(nu)
                    gather_start(nu)

                write_start(i, u)

        write_wait(0)
        write_wait(1)

    return gather


def kernel(sequences_tokens, embedding_table):
    b, l = sequences_tokens.shape
    v, d = embedding_table.shape
    idx = sequences_tokens.reshape(b * l)
    av = jnp.arange(2)[:, None, None]
    bv = jnp.arange(d)[None, :, None]
    cv = jnp.arange(2 * d)[None, None, :]
    pack_onehot = (cv == av * d + bv).astype(jnp.float32)
    t2 = lax.dot_general(
        embedding_table.reshape(v // 2, 2, d), pack_onehot,
        (((1, 2), (0, 1)), ((), ())),
        preferred_element_type=jnp.float32)
    t_view = t2.reshape(v, d)
    out = _build_gather(b * l, d)(idx, t_view)
    eye = jnp.eye(d, dtype=jnp.float32)
    y = out.reshape(b, l, d)
    return lax.dot_general(
        y, eye, (((2,), (0,)), ((), ())),
        preferred_element_type=jnp.float32)
